# baseline (speedup 1.0000x reference)
import functools

import jax
import jax.numpy as jnp
from jax.experimental import pallas as pl
from jax.experimental.pallas import tpu as pltpu

_NQ = 2048
_C = 128
_NB = 50000
_TA = 1000
_TB = 2000
_K = 9
_HW = 1024


def _min_kernel(mb1_ref, mb2_ref, qt_ref, o_ref, acc_ref):
    j = pl.program_id(0)
    s1 = jnp.dot(mb1_ref[...], qt_ref[...], preferred_element_type=jnp.float32)
    mn1 = jnp.sum(mb1_ref[...] * mb1_ref[...], axis=1, keepdims=True)
    s2 = jnp.dot(mb2_ref[...], qt_ref[...], preferred_element_type=jnp.float32)
    mn2 = jnp.sum(mb2_ref[...] * mb2_ref[...], axis=1, keepdims=True)
    d = jnp.minimum(mn1 - s1, mn2 - s2)
    m = jnp.min(d, axis=0, keepdims=True)

    @pl.when(j == 0)
    def _():
        acc_ref[...] = m

    @pl.when(j > 0)
    def _():
        acc_ref[...] = jnp.minimum(acc_ref[...], m)

    @pl.when(j == pl.num_programs(0) - 1)
    def _():
        qn = 0.25 * jnp.sum(qt_ref[...] * qt_ref[...], axis=0, keepdims=True)
        o_ref[...] = jnp.sqrt(jnp.maximum(acc_ref[...] + qn, 0.0))


def _topk_kernel(mb1_ref, mb2_ref, q_ref, pix_ref, o_ref, top_ref):
    j = pl.program_id(0)

    @pl.when(j == 0)
    def _():
        top_ref[...] = jnp.full((8, 128), jnp.inf, jnp.float32)

    g0 = jnp.argmax(pix_ref[0:1, 0:_HW])
    g1 = _HW + jnp.argmax(pix_ref[0:1, _HW:2 * _HW])
    qs = jnp.concatenate(
        [q_ref[pl.ds(g0, 1), :], q_ref[pl.ds(g1, 1), :],
         jnp.zeros((6, _C), jnp.float32)], axis=0)

    dims = (((1,), (1,)), ((), ()))
    ones = jnp.ones((8, _C), jnp.float32)
    s1 = jax.lax.dot_general(qs, mb1_ref[...], dims,
                             preferred_element_type=jnp.float32)
    mnt1 = jax.lax.dot_general(ones, mb1_ref[...] * mb1_ref[...], dims,
                               preferred_element_type=jnp.float32)
    s2 = jax.lax.dot_general(qs, mb2_ref[...], dims,
                             preferred_element_type=jnp.float32)
    mnt2 = jax.lax.dot_general(ones, mb2_ref[...] * mb2_ref[...], dims,
                               preferred_element_type=jnp.float32)
    d1 = mnt1 - s1
    d2 = jnp.where(j < pl.num_programs(0) - 1, mnt2 - s2, jnp.inf)

    cand = jnp.concatenate([top_ref[...], d1, d2], axis=1)
    lanes = jax.lax.broadcasted_iota(jnp.int32, cand.shape, 1)
    out_lane = lanes[:, 0:128]
    newtop = jnp.full((8, 128), jnp.inf, jnp.float32)
    for k in range(_K):
        mv = jnp.min(cand, axis=1, keepdims=True)
        am = jnp.argmin(cand, axis=1)
        cand = jnp.where(lanes == am[:, None], jnp.inf, cand)
        newtop = jnp.where(out_lane == k, mv, newtop)
    top_ref[...] = newtop

    @pl.when(j == pl.num_programs(0) - 1)
    def _():
        qn = 0.25 * jnp.sum(qs * qs, axis=1, keepdims=True)
        t9 = jnp.maximum(top_ref[...] + qn, 0.0)
        sa = jnp.sqrt(t9)
        valid = out_lane < _K
        mx = jnp.max(jnp.where(valid, sa, -jnp.inf), axis=1, keepdims=True)
        e = jnp.where(valid, jnp.exp(sa - mx), 0.0)
        ssum = jnp.sum(e, axis=1, keepdims=True)
        sm0 = e[:, 0:1] / ssum
        img = sa[:, 0:1] * (1.0 - sm0)
        o_ref[...] = jnp.broadcast_to(img, (8, 128))


@functools.partial(jax.jit, static_argnames=())
def kernel(feature_batch, memory_bank):
    B, H, W, C = feature_batch.shape
    fv2 = 2.0 * feature_batch.reshape(B * H * W, C)
    qt2 = fv2.T

    pix = pl.pallas_call(
        _min_kernel,
        grid=(_NB // _TA // 2,),
        in_specs=[
            pl.BlockSpec((_TA, _C), lambda j: (2 * j, 0)),
            pl.BlockSpec((_TA, _C), lambda j: (2 * j + 1, 0)),
            pl.BlockSpec((_C, _NQ), lambda j: (0, 0)),
        ],
        out_specs=pl.BlockSpec((1, _NQ), lambda j: (0, 0)),
        out_shape=jax.ShapeDtypeStruct((1, _NQ), jnp.float32),
        scratch_shapes=[pltpu.VMEM((1, _NQ), jnp.float32)],
    )(memory_bank, memory_bank, qt2)

    pixel_scores = pix.reshape(B, 1, H, W)

    nb_tiles = _NB // _TB
    img8 = pl.pallas_call(
        _topk_kernel,
        grid=((nb_tiles + 1) // 2,),
        in_specs=[
            pl.BlockSpec((_TB, _C), lambda j: (2 * j, 0)),
            pl.BlockSpec((_TB, _C),
                         lambda j: (jnp.minimum(2 * j + 1, nb_tiles - 1), 0)),
            pl.BlockSpec((_NQ, _C), lambda j: (0, 0)),
            pl.BlockSpec((1, _NQ), lambda j: (0, 0)),
        ],
        out_specs=pl.BlockSpec((8, 128), lambda j: (0, 0)),
        out_shape=jax.ShapeDtypeStruct((8, 128), jnp.float32),
        scratch_shapes=[pltpu.VMEM((8, 128), jnp.float32)],
    )(memory_bank, memory_bank, fv2, pix)

    image_scores = img8[0:B, 0]
    return (pixel_scores, image_scores)


# speedup vs baseline: 1.0867x; 1.0867x over previous
import functools

import jax
import jax.numpy as jnp
from jax.experimental import pallas as pl
from jax.experimental.pallas import tpu as pltpu

_NQ = 2048
_C = 128
_NB = 50000
_TA = 1000
_TB = 2000
_K = 9
_HW = 1024


def _min_kernel(mb1_ref, mb2_ref, qt_ref, o_ref, acc_ref):
    j = pl.program_id(0)
    s1 = jnp.dot(mb1_ref[...], qt_ref[...], preferred_element_type=jnp.float32)
    mn1 = jnp.sum(mb1_ref[...] * mb1_ref[...], axis=1, keepdims=True)
    s2 = jnp.dot(mb2_ref[...], qt_ref[...], preferred_element_type=jnp.float32)
    mn2 = jnp.sum(mb2_ref[...] * mb2_ref[...], axis=1, keepdims=True)
    d = jnp.minimum(mn1 - s1, mn2 - s2)
    m = jnp.min(d, axis=0, keepdims=True)

    @pl.when(j == 0)
    def _():
        acc_ref[...] = m

    @pl.when(j > 0)
    def _():
        acc_ref[...] = jnp.minimum(acc_ref[...], m)

    @pl.when(j == pl.num_programs(0) - 1)
    def _():
        qn = 0.25 * jnp.sum(qt_ref[...] * qt_ref[...], axis=0, keepdims=True)
        o_ref[...] = jnp.sqrt(jnp.maximum(acc_ref[...] + qn, 0.0))


def _topk_kernel(mb1_ref, mb2_ref, mb3_ref, mb4_ref, q_ref, pix_ref, o_ref,
                 top_ref):
    j = pl.program_id(0)

    @pl.when(j == 0)
    def _():
        top_ref[...] = jnp.full((8, 128), jnp.inf, jnp.float32)

    g0 = jnp.argmax(pix_ref[0:1, 0:_HW])
    g1 = _HW + jnp.argmax(pix_ref[0:1, _HW:2 * _HW])
    qs = jnp.concatenate(
        [q_ref[pl.ds(g0, 1), :], q_ref[pl.ds(g1, 1), :],
         jnp.zeros((6, _C), jnp.float32)], axis=0)

    dims = (((1,), (1,)), ((), ()))
    ones = jnp.ones((8, _C), jnp.float32)
    nt = _NB // _TB
    parts = []
    for c, ref in enumerate((mb1_ref, mb2_ref, mb3_ref, mb4_ref)):
        s = jax.lax.dot_general(qs, ref[...], dims,
                                preferred_element_type=jnp.float32)
        mnt = jax.lax.dot_general(ones, ref[...] * ref[...], dims,
                                  preferred_element_type=jnp.float32)
        d = mnt - s
        if c > 0:
            d = jnp.where(4 * j + c <= nt - 1, d, jnp.inf)
        parts.append(d)

    cand = jnp.concatenate([top_ref[...]] + parts, axis=1)
    lanes = jax.lax.broadcasted_iota(jnp.int32, cand.shape, 1)
    out_lane = lanes[:, 0:128]
    newtop = jnp.full((8, 128), jnp.inf, jnp.float32)
    for k in range(_K):
        mv = jnp.min(cand, axis=1, keepdims=True)
        am = jnp.argmin(cand, axis=1)
        cand = jnp.where(lanes == am[:, None], jnp.inf, cand)
        newtop = jnp.where(out_lane == k, mv, newtop)
    top_ref[...] = newtop

    @pl.when(j == pl.num_programs(0) - 1)
    def _():
        qn = 0.25 * jnp.sum(qs * qs, axis=1, keepdims=True)
        t9 = jnp.maximum(top_ref[...] + qn, 0.0)
        sa = jnp.sqrt(t9)
        valid = out_lane < _K
        mx = jnp.max(jnp.where(valid, sa, -jnp.inf), axis=1, keepdims=True)
        e = jnp.where(valid, jnp.exp(sa - mx), 0.0)
        ssum = jnp.sum(e, axis=1, keepdims=True)
        sm0 = e[:, 0:1] / ssum
        img = sa[:, 0:1] * (1.0 - sm0)
        o_ref[...] = jnp.broadcast_to(img, (8, 128))


@functools.partial(jax.jit, static_argnames=())
def kernel(feature_batch, memory_bank):
    B, H, W, C = feature_batch.shape
    fv2 = 2.0 * feature_batch.reshape(B * H * W, C)
    qt2 = fv2.T

    pix = pl.pallas_call(
        _min_kernel,
        grid=(_NB // _TA // 2,),
        in_specs=[
            pl.BlockSpec((_TA, _C), lambda j: (2 * j, 0)),
            pl.BlockSpec((_TA, _C), lambda j: (2 * j + 1, 0)),
            pl.BlockSpec((_C, _NQ), lambda j: (0, 0)),
        ],
        out_specs=pl.BlockSpec((1, _NQ), lambda j: (0, 0)),
        out_shape=jax.ShapeDtypeStruct((1, _NQ), jnp.float32),
        scratch_shapes=[pltpu.VMEM((1, _NQ), jnp.float32)],
    )(memory_bank, memory_bank, qt2)

    pixel_scores = pix.reshape(B, 1, H, W)

    nb_tiles = _NB // _TB
    img8 = pl.pallas_call(
        _topk_kernel,
        grid=((nb_tiles + 3) // 4,),
        in_specs=[
            pl.BlockSpec((_TB, _C), lambda j: (4 * j, 0)),
            pl.BlockSpec((_TB, _C),
                         lambda j: (jnp.minimum(4 * j + 1, nb_tiles - 1), 0)),
            pl.BlockSpec((_TB, _C),
                         lambda j: (jnp.minimum(4 * j + 2, nb_tiles - 1), 0)),
            pl.BlockSpec((_TB, _C),
                         lambda j: (jnp.minimum(4 * j + 3, nb_tiles - 1), 0)),
            pl.BlockSpec((_NQ, _C), lambda j: (0, 0)),
            pl.BlockSpec((1, _NQ), lambda j: (0, 0)),
        ],
        out_specs=pl.BlockSpec((8, 128), lambda j: (0, 0)),
        out_shape=jax.ShapeDtypeStruct((8, 128), jnp.float32),
        scratch_shapes=[pltpu.VMEM((8, 128), jnp.float32)],
    )(memory_bank, memory_bank, memory_bank, memory_bank, fv2, pix)

    image_scores = img8[0:B, 0]
    return (pixel_scores, image_scores)
